# baseline (speedup 1.0000x reference)
import functools

import jax
import jax.numpy as jnp
from jax import lax
from jax.experimental import pallas as pl
from jax.experimental.pallas import tpu as pltpu
from jax.experimental.pallas import tpu_sc as plsc

N = 16384
D = 64
_INFO = plsc.get_sparse_core_info()
_NC, _NS = _INFO.num_cores, _INFO.num_subcores
_NW = _NC * _NS
_BPW = N // _NW
_L = 16

_MESH = plsc.VectorSubcoreMesh(core_axis_name="c", subcore_axis_name="s")


_CHUNK = 256


def _gather_one_table(pairs_hbm, idx_v, idx2_v, pair_v, out_v, sem):
    iota = lax.iota(jnp.int32, _L)
    for t in range(_BPW // _CHUNK):
        toff = t * _CHUNK
        for i in range(_CHUNK // _L):
            sl = pl.ds(toff + i * _L, _L)
            idx2_v[pl.ds(i * _L, _L)] = idx_v[sl] >> 1
        pltpu.async_copy(pairs_hbm.at[idx2_v], pair_v, sem).wait()

        def body(c, _, toff=toff):
            rows_l = c * _L + iota
            rows_g = toff + rows_l
            h = (idx_v[pl.ds(toff + c * _L, _L)] & 1) * D
            for j in range(D):
                vals = plsc.load_gather(pair_v, [rows_l, h + j])
                plsc.store_scatter(out_v,
                                   [rows_g, jnp.full((_L,), j, jnp.int32)],
                                   vals)
            return _

        lax.fori_loop(0, _CHUNK // _L, body, None)


@functools.partial(
    pl.kernel,
    mesh=_MESH,
    compiler_params=pltpu.CompilerParams(needs_layout_passes=False),
    out_type=[
        jax.ShapeDtypeStruct((N, D), jnp.float32),
        jax.ShapeDtypeStruct((N, D), jnp.float32),
    ],
    scratch_types=[
        pltpu.VMEM((_BPW,), jnp.int32),
        pltpu.VMEM((_BPW,), jnp.int32),
        pltpu.VMEM((_CHUNK,), jnp.int32),
        pltpu.VMEM((_CHUNK, 2 * D), jnp.float32),
        pltpu.VMEM((_BPW, D), jnp.float32),
        pltpu.SemaphoreType.DMA,
    ],
)
def _sc_gather(port_pairs, port_idx, proto_pairs, proto_idx,
               out_port, out_proto,
               pidx_v, qidx_v, idx2_v, pair_v, out_v, sem):
    wid = lax.axis_index("s") * _NC + lax.axis_index("c")
    base = wid * _BPW
    pltpu.sync_copy(port_idx.at[pl.ds(base, _BPW)], pidx_v)
    pltpu.sync_copy(proto_idx.at[pl.ds(base, _BPW)], qidx_v)

    _gather_one_table(port_pairs, pidx_v, idx2_v, pair_v, out_v, sem)
    pltpu.sync_copy(out_v, out_port.at[pl.ds(base, _BPW)])

    _gather_one_table(proto_pairs, qidx_v, idx2_v, pair_v, out_v, sem)
    pltpu.sync_copy(out_v, out_proto.at[pl.ds(base, _BPW)])


def _ip_body(x_ref, w_ref, b_ref, o_ref):
    o_ref[...] = (
        jnp.dot(x_ref[...], w_ref[...], preferred_element_type=jnp.float32)
        + b_ref[...]
    )


_IP_BLK = 2048


def _ip_linear(ip_features, W_ip, b_ip):
    return pl.pallas_call(
        _ip_body,
        grid=(N // _IP_BLK,),
        in_specs=[
            pl.BlockSpec((_IP_BLK, 32), lambda i: (i, 0)),
            pl.BlockSpec((32, D), lambda i: (0, 0)),
            pl.BlockSpec((1, D), lambda i: (0, 0)),
        ],
        out_specs=pl.BlockSpec((_IP_BLK, D), lambda i: (i, 0)),
        out_shape=jax.ShapeDtypeStruct((N, D), jnp.float32),
    )(ip_features, W_ip, b_ip.reshape(1, D))


def kernel(ip_features, port_indices, proto_indices, W_ip, b_ip,
           port_table, proto_table):
    x_ip = _ip_linear(ip_features, W_ip, b_ip)
    port_pairs = jnp.reshape(port_table, (port_table.shape[0] // 2, 2 * D))
    proto_pairs = jnp.reshape(proto_table, (proto_table.shape[0] // 2, 2 * D))
    x_port, x_proto = _sc_gather(
        port_pairs, port_indices.astype(jnp.int32),
        proto_pairs, proto_indices.astype(jnp.int32))
    return (x_ip, x_port, x_proto)


# speedup vs baseline: 1.8868x; 1.8868x over previous
import functools

import jax
import jax.numpy as jnp
from jax import lax
from jax.experimental import pallas as pl
from jax.experimental.pallas import tpu as pltpu
from jax.experimental.pallas import tpu_sc as plsc

N = 16384
D = 64
_PORT_V = 65536
_PROTO_V = 256
_INFO = plsc.get_sparse_core_info()
_NC, _NS = _INFO.num_cores, _INFO.num_subcores
_NW = _NC * _NS
_BPW = N // _NW
_CHUNK = 256

_MESH = plsc.VectorSubcoreMesh(core_axis_name="c", subcore_axis_name="s")


@functools.partial(
    pl.kernel,
    mesh=_MESH,
    compiler_params=pltpu.CompilerParams(needs_layout_passes=False),
    out_type=[
        jax.ShapeDtypeStruct((N, 2 * D), jnp.float32),
        jax.ShapeDtypeStruct((N, 2 * D), jnp.float32),
    ],
    scratch_types=[
        [pltpu.VMEM((_CHUNK,), jnp.int32) for _ in range(4)],
        [pltpu.VMEM((_CHUNK, 2 * D), jnp.float32) for _ in range(2)],
        [pltpu.SemaphoreType.DMA for _ in range(2)],
        [pltpu.SemaphoreType.DMA for _ in range(2)],
    ],
)
def _sc_gather(port128, port_idx, proto128, proto_idx,
               out_port, out_proto,
               idx_bufs, pair_bufs, gsems, wsems):
    wid = lax.axis_index("s") * _NC + lax.axis_index("c")
    base = wid * _BPW

    chunks = []
    for t in range(_BPW // _CHUNK):
        chunks.append((port_idx, port128, out_port, t * _CHUNK))
    for t in range(_BPW // _CHUNK):
        chunks.append((proto_idx, proto128, out_proto, t * _CHUNK))

    for c, (idx_hbm, _, _, toff) in enumerate(chunks):
        pltpu.sync_copy(idx_hbm.at[pl.ds(base + toff, _CHUNK)], idx_bufs[c])

    n = len(chunks)
    gathers = [None] * n
    writes = [None] * n

    def start_gather(c):
        table = chunks[c][1]
        return pltpu.async_copy(
            table.at[idx_bufs[c]], pair_bufs[c % 2], gsems[c % 2])

    gathers[0] = start_gather(0)
    gathers[1] = start_gather(1)
    for c, (_, _, out, toff) in enumerate(chunks):
        gathers[c].wait()
        writes[c] = pltpu.async_copy(
            pair_bufs[c % 2], out.at[pl.ds(base + toff, _CHUNK)],
            wsems[c % 2])
        if c + 2 < n:
            writes[c].wait()
            gathers[c + 2] = start_gather(c + 2)
    writes[n - 2].wait()
    writes[n - 1].wait()


_TBLK = 2048


def _tp_body(x_ref, o_ref):
    o_ref[:, 0:D] = x_ref[...].T


def _row_major_padded(table_t, vocab):
    blk = min(_TBLK, vocab)
    return pl.pallas_call(
        _tp_body,
        grid=(vocab // blk,),
        in_specs=[pl.BlockSpec((D, blk), lambda i: (0, i))],
        out_specs=pl.BlockSpec((blk, 2 * D), lambda i: (i, 0)),
        out_shape=jax.ShapeDtypeStruct((vocab, 2 * D), jnp.float32),
    )(table_t)


def _ip_body(w_ref, x_ref, b_ref, o_ref):
    o_ref[...] = (
        lax.dot_general(w_ref[...], x_ref[...], (((0,), (0,)), ((), ())),
                        preferred_element_type=jnp.float32)
        + b_ref[...]
    )


_IP_BLK = 4096


def _ip_linear(ip_features_t, W_ip, b_ip):
    return pl.pallas_call(
        _ip_body,
        grid=(N // _IP_BLK,),
        in_specs=[
            pl.BlockSpec((32, D), lambda i: (0, 0)),
            pl.BlockSpec((32, _IP_BLK), lambda i: (0, i)),
            pl.BlockSpec((D, 1), lambda i: (0, 0)),
        ],
        out_specs=pl.BlockSpec((D, _IP_BLK), lambda i: (0, i)),
        out_shape=jax.ShapeDtypeStruct((D, N), jnp.float32),
    )(W_ip, ip_features_t, b_ip.reshape(D, 1))


def kernel(ip_features, port_indices, proto_indices, W_ip, b_ip,
           port_table, proto_table):
    x_ip_t = _ip_linear(ip_features.T, W_ip, b_ip)
    port128 = _row_major_padded(port_table.T, _PORT_V)
    proto128 = _row_major_padded(proto_table.T, _PROTO_V)
    xp128, xq128 = _sc_gather(
        port128, port_indices.astype(jnp.int32),
        proto128, proto_indices.astype(jnp.int32))
    return (x_ip_t.T, xp128[:, :D], xq128[:, :D])


# speedup vs baseline: 4.3475x; 2.3042x over previous
import functools

import jax
import jax.numpy as jnp
from jax import lax
from jax.experimental import pallas as pl
from jax.experimental.pallas import tpu as pltpu
from jax.experimental.pallas import tpu_sc as plsc

N = 16384
D = 64
_PORT_V = 65536
_PROTO_V = 256
_INFO = plsc.get_sparse_core_info()
_NC, _NS = _INFO.num_cores, _INFO.num_subcores
_NW = _NC * _NS
_BPW = N // _NW
_CHUNK = 128
_NBUF = 4

_MESH = plsc.VectorSubcoreMesh(core_axis_name="c", subcore_axis_name="s")


@functools.partial(
    pl.kernel,
    mesh=_MESH,
    compiler_params=pltpu.CompilerParams(needs_layout_passes=False),
    out_type=[
        jax.ShapeDtypeStruct((N, 2 * D), jnp.float32),
        jax.ShapeDtypeStruct((N, 2 * D), jnp.float32),
    ],
    scratch_types=[
        [pltpu.VMEM((_CHUNK,), jnp.int32) for _ in range(2 * _BPW // _CHUNK)],
        [pltpu.VMEM((_CHUNK, 2 * D), jnp.float32) for _ in range(_NBUF)],
        [pltpu.SemaphoreType.DMA for _ in range(_NBUF)],
        [pltpu.SemaphoreType.DMA for _ in range(_NBUF)],
    ],
)
def _sc_gather(port128, port_idx, proto128, proto_idx,
               out_port, out_proto,
               idx_bufs, pair_bufs, gsems, wsems):
    wid = lax.axis_index("s") * _NC + lax.axis_index("c")
    base = wid * _BPW

    chunks = []
    for t in range(_BPW // _CHUNK):
        chunks.append((port_idx, port128, out_port, t * _CHUNK))
    for t in range(_BPW // _CHUNK):
        chunks.append((proto_idx, proto128, out_proto, t * _CHUNK))

    for c, (idx_hbm, _, _, toff) in enumerate(chunks):
        pltpu.sync_copy(idx_hbm.at[pl.ds(base + toff, _CHUNK)], idx_bufs[c])

    n = len(chunks)
    gathers = [None] * n
    writes = [None] * n
    lookahead = _NBUF - 1

    def start_gather(c):
        table = chunks[c][1]
        return pltpu.async_copy(
            table.at[idx_bufs[c]], pair_bufs[c % _NBUF], gsems[c % _NBUF])

    for c in range(lookahead):
        gathers[c] = start_gather(c)
    for c, (_, _, out, toff) in enumerate(chunks):
        gathers[c].wait()
        pltpu.sync_copy(pair_bufs[c % _NBUF],
                        out.at[pl.ds(base + toff, _CHUNK)])
        if c + lookahead < n:
            gathers[c + lookahead] = start_gather(c + lookahead)


_TBLK = 2048


def _tp_body(x_ref, o_ref):
    o_ref[:, 0:D] = x_ref[...].T


def _row_major_padded(table_t, vocab):
    blk = min(_TBLK, vocab)
    return pl.pallas_call(
        _tp_body,
        grid=(vocab // blk,),
        in_specs=[pl.BlockSpec((D, blk), lambda i: (0, i))],
        out_specs=pl.BlockSpec((blk, 2 * D), lambda i: (i, 0)),
        out_shape=jax.ShapeDtypeStruct((vocab, 2 * D), jnp.float32),
    )(table_t)


def _ip_body(w_ref, x_ref, b_ref, o_ref):
    o_ref[...] = (
        lax.dot_general(w_ref[...], x_ref[...], (((0,), (0,)), ((), ())),
                        preferred_element_type=jnp.float32)
        + b_ref[...]
    )


_IP_BLK = 4096


def _ip_linear(ip_features_t, W_ip, b_ip):
    return pl.pallas_call(
        _ip_body,
        grid=(N // _IP_BLK,),
        in_specs=[
            pl.BlockSpec((32, D), lambda i: (0, 0)),
            pl.BlockSpec((32, _IP_BLK), lambda i: (0, i)),
            pl.BlockSpec((D, 1), lambda i: (0, 0)),
        ],
        out_specs=pl.BlockSpec((D, _IP_BLK), lambda i: (0, i)),
        out_shape=jax.ShapeDtypeStruct((D, N), jnp.float32),
    )(W_ip, ip_features_t, b_ip.reshape(D, 1))


def kernel(ip_features, port_indices, proto_indices, W_ip, b_ip,
           port_table, proto_table):
    x_ip_t = _ip_linear(ip_features.T, W_ip, b_ip)
    port128 = _row_major_padded(port_table.T, _PORT_V)
    proto128 = _row_major_padded(proto_table.T, _PROTO_V)
    xp128, xq128 = _sc_gather(
        port128, port_indices.astype(jnp.int32),
        proto128, proto_indices.astype(jnp.int32))
    return (x_ip_t.T, xp128[:, :D], xq128[:, :D])
